# Initial kernel scaffold; baseline (speedup 1.0000x reference)
import functools

import jax
import jax.numpy as jnp
from jax import lax
from jax.experimental import pallas as pl
from jax.experimental.pallas import tpu as pltpu
from jax.experimental.pallas import tpu_sc as plsc

_BATCH = 16384
_HIST = 50
_DIM = 64
_B = _BATCH * _HIST

_NC = 2
_NS = 16
_NW = _NC * _NS
_B_PER_W = _B // _NW

_CHUNK = 128
_CHUNKS_PER_W = _B_PER_W // _CHUNK
_GROUP = 4
_ROWS_PER_GROUP = _GROUP * _CHUNK
_N_GROUPS = _CHUNKS_PER_W // _GROUP


def _emb_body(idx_hbm, table_hbm, out_hbm, idx_v, rows0, rows1, sem0, sem1):
    cid = lax.axis_index("c")
    sid = lax.axis_index("s")
    wid = sid * _NC + cid
    cbase = wid * _CHUNKS_PER_W
    rbase = wid * _B_PER_W

    pltpu.sync_copy(idx_hbm.at[pl.ds(cbase, _CHUNKS_PER_W)], idx_v)

    def issue(g, buf, sem):
        for j in range(_GROUP):
            pltpu.async_copy(
                table_hbm.at[idx_v.at[g * _GROUP + j]],
                buf.at[pl.ds(j * _CHUNK, _CHUNK)],
                sem,
            )

    def drain_write(g, buf, sem):
        pltpu.make_async_copy(
            table_hbm.at[pl.ds(0, _ROWS_PER_GROUP)], buf, sem
        ).wait()
        pltpu.sync_copy(
            buf, out_hbm.at[pl.ds(rbase + g * _ROWS_PER_GROUP, _ROWS_PER_GROUP)]
        )

    issue(0, rows0, sem0)
    issue(1, rows1, sem1)

    @functools.partial(pl.loop, 0, _N_GROUPS - 2, step=2)
    def _(g):
        drain_write(g, rows0, sem0)
        issue(g + 2, rows0, sem0)
        drain_write(g + 1, rows1, sem1)
        issue(g + 3, rows1, sem1)

    drain_write(_N_GROUPS - 2, rows0, sem0)
    drain_write(_N_GROUPS - 1, rows1, sem1)


@jax.jit
def kernel(token_ids, weights):
    idx = token_ids.astype(jnp.int32).reshape(_NW * _CHUNKS_PER_W, _CHUNK)
    run = pl.kernel(
        _emb_body,
        out_type=jax.ShapeDtypeStruct((_B, _DIM), jnp.float32),
        mesh=plsc.VectorSubcoreMesh(core_axis_name="c", subcore_axis_name="s"),
        scratch_types=[
            pltpu.VMEM((_CHUNKS_PER_W, _CHUNK), jnp.int32),
            pltpu.VMEM((_ROWS_PER_GROUP, _DIM), jnp.float32),
            pltpu.VMEM((_ROWS_PER_GROUP, _DIM), jnp.float32),
            pltpu.SemaphoreType.DMA,
            pltpu.SemaphoreType.DMA,
        ],
    )
    out = run(idx, weights)
    return out.reshape(_BATCH, _HIST, _DIM)


# speedup vs baseline: 1.8762x; 1.8762x over previous
import functools

import jax
import jax.numpy as jnp
from jax import lax
from jax.experimental import pallas as pl
from jax.experimental.pallas import tpu as pltpu
from jax.experimental.pallas import tpu_sc as plsc

_BATCH = 16384
_HIST = 50
_DIM = 64
_B = _BATCH * _HIST

_NC = 2
_NS = 16
_NW = _NC * _NS
_B_PER_W = _B // _NW

_CHUNK = 128
_CHUNKS_PER_W = _B_PER_W // _CHUNK
_GROUP = 4
_ROWS_PER_GROUP = _GROUP * _CHUNK
_N_GROUPS = _CHUNKS_PER_W // _GROUP


def _emb_body(idx_hbm, table_hbm, out_hbm, idx_v, rows0, rows1, sem0, sem1):
    cid = lax.axis_index("c")
    sid = lax.axis_index("s")
    wid = sid * _NC + cid
    cbase = wid * _CHUNKS_PER_W
    rbase = wid * _B_PER_W

    pltpu.sync_copy(idx_hbm.at[pl.ds(cbase, _CHUNKS_PER_W)], idx_v)

    def issue(g, buf, sem):
        for j in range(_GROUP):
            pltpu.async_copy(
                table_hbm.at[idx_v.at[g * _GROUP + j]],
                buf.at[pl.ds(j * _CHUNK, _CHUNK)],
                sem,
            )

    def drain_write(g, buf, sem):
        pltpu.make_async_copy(
            table_hbm.at[pl.ds(0, _ROWS_PER_GROUP)], buf, sem
        ).wait()
        pltpu.sync_copy(
            buf, out_hbm.at[pl.ds(rbase + g * _ROWS_PER_GROUP, _ROWS_PER_GROUP)]
        )

    issue(0, rows0, sem0)
    issue(1, rows1, sem1)

    @pl.loop(0, _N_GROUPS - 2, step=2)
    def _(g):
        drain_write(g, rows0, sem0)
        issue(g + 2, rows0, sem0)
        drain_write(g + 1, rows1, sem1)
        issue(g + 3, rows1, sem1)

    drain_write(_N_GROUPS - 2, rows0, sem0)
    drain_write(_N_GROUPS - 1, rows1, sem1)


@jax.jit
def kernel(token_ids, weights):
    idx = token_ids.astype(jnp.int32).reshape(_NW * _CHUNKS_PER_W, _CHUNK)
    run = pl.kernel(
        _emb_body,
        out_type=jax.ShapeDtypeStruct((_B, _DIM), jnp.float32),
        mesh=plsc.VectorSubcoreMesh(core_axis_name="c", subcore_axis_name="s"),
        scratch_types=[
            pltpu.VMEM((_CHUNKS_PER_W, _CHUNK), jnp.int32),
            pltpu.VMEM((_ROWS_PER_GROUP, _DIM), jnp.float32),
            pltpu.VMEM((_ROWS_PER_GROUP, _DIM), jnp.float32),
            pltpu.SemaphoreType.DMA,
            pltpu.SemaphoreType.DMA,
        ],
        compiler_params=pltpu.CompilerParams(use_tc_tiling_on_sc=False),
    )
    out = run(idx, weights)
    return out.reshape(_BATCH, _HIST, _DIM)
